# baseline (speedup 1.0000x reference)
import functools

import jax
import jax.numpy as jnp
from jax import lax
from jax.experimental import pallas as pl
from jax.experimental.pallas import tpu as pltpu
from jax.experimental.pallas import tpu_sc as plsc

B = 16
B_H = 8
N = 20000
FEAT = 224
W = 256
HROW = 225
ROWS_P = 232
ACC = ROWS_P * W
NC = 2
NS = 16
NPAD = 20480
QBOX = NPAD // 4
CH = 256
NCHUNK = QBOX // CH
SUB = 32
NSUB = CH // SUB
ZB = ACC // 8


def _make_sc_body(boff):
  def _sc_scatter_body(conf, codes, out, acc, cb_a, qb_a, cb_b, qb_b, ib_a,
                       vb_a, ib_b, vb_b, zb_a, zb_b, semi_a, semi_b, sems_a,
                       sems_b, sem_z, sem_o):
    c = lax.axis_index("c")
    s = lax.axis_index("s")
    b_loc = s // 2
    quarter = 2 * (s % 2) + c
    soff = b_loc * ACC
    base = (boff + b_loc) * NPAD + quarter * QBOX
    m8 = jnp.int32(255)
    m16 = jnp.int32(0xFFFF)
    mrow = jnp.int32(0xFF00)

    def _stage(t, cbr, qbr, sem):
      st = base + t * CH
      pltpu.async_copy(conf.at[pl.ds(st, CH)], cbr, sem)
      pltpu.async_copy(codes.at[pl.ds(st, CH)], qbr, sem)

    def _drain_stage(cbr, qbr, sem):
      pltpu.make_async_copy(conf.at[pl.ds(0, CH)], cbr, sem).wait()
      pltpu.make_async_copy(codes.at[pl.ds(0, CH)], qbr, sem).wait()

    def _fill(cbr, qbr, ibr, vbr):
      for j in range(NSUB):
        for g in range(SUB // 16):
          og = j * SUB + g * 16
          cf = cbr[pl.ds(og, 16)]
          q = qbr[pl.ds(og, 16)]
          qh = (q >> 16) & m16
          ql = q & m16
          off = g * 64
          ibr[j, pl.ds(off, 16)] = soff + ql
          ibr[j, pl.ds(off + 16, 16)] = soff + ((ql & mrow) | (qh & m8))
          ibr[j, pl.ds(off + 32, 16)] = soff + ((qh & mrow) | (ql & m8))
          ibr[j, pl.ds(off + 48, 16)] = soff + qh
          ncf = -cf
          vbr[j, pl.ds(off, 16)] = cf
          vbr[j, pl.ds(off + 16, 16)] = ncf
          vbr[j, pl.ds(off + 32, 16)] = ncf
          vbr[j, pl.ds(off + 48, 16)] = cf

    def _fire(ibr, vbr, sem):
      for j in range(NSUB):
        pltpu.async_copy(vbr.at[j], acc.at[ibr.at[j]], sem, add=True)

    def _drain_scat(vbr, sem):
      for j in range(NSUB):
        pltpu.make_async_copy(vbr.at[j], acc.at[pl.ds(0, 4 * SUB)],
                              sem).wait()

    _stage(0, cb_a, qb_a, semi_a)

    def _zb(i, carry):
      zb_a[pl.ds(i * 16, 16)] = jnp.zeros((16,), jnp.float32)
      return carry

    lax.fori_loop(0, ZB // 16, _zb, 0)

    @pl.when(s % 2 == 0)
    def _():
      zds = [
          pltpu.async_copy(zb_a, acc.at[pl.ds(soff + k * ZB, ZB)], sem_z)
          for k in range(ACC // ZB)
      ]
      for d in zds:
        d.wait()

    plsc.subcore_barrier()

    def _pair(u, carry):
      _stage(2 * u + 1, cb_b, qb_b, semi_b)
      _drain_stage(cb_a, qb_a, semi_a)

      @pl.when(u > 0)
      def _():
        _drain_scat(vb_a, sems_a)

      _fill(cb_a, qb_a, ib_a, vb_a)
      _fire(ib_a, vb_a, sems_a)
      _drain_stage(cb_b, qb_b, semi_b)

      @pl.when(u > 0)
      def _():
        _drain_scat(vb_b, sems_b)

      _fill(cb_b, qb_b, ib_b, vb_b)
      _fire(ib_b, vb_b, sems_b)

      @pl.when(u + 1 < NCHUNK // 2)
      def _():
        _stage(2 * u + 2, cb_a, qb_a, semi_a)

      return carry

    lax.fori_loop(0, NCHUNK // 2, _pair, 0)
    _drain_scat(vb_a, sems_a)
    _drain_scat(vb_b, sems_b)

    plsc.subcore_barrier()

    @pl.when(s % 2 == 0)
    def _():
      obase = (c * B_H + b_loc) * ACC
      nco = ACC // ZB
      zb = (zb_a, zb_b)
      dins = [None] * nco
      douts = [None] * nco
      dins[0] = pltpu.async_copy(acc.at[pl.ds(soff, ZB)], zb_a, sem_z)
      for k in range(nco):
        dins[k].wait()
        douts[k] = pltpu.async_copy(
            zb[k % 2], out.at[pl.ds(obase + k * ZB, ZB)], sem_o)
        if k + 1 < nco:
          if k >= 1:
            douts[k - 1].wait()
          dins[k + 1] = pltpu.async_copy(
              acc.at[pl.ds(soff + (k + 1) * ZB, ZB)], zb[(k + 1) % 2], sem_z)
      douts[nco - 2].wait()
      douts[nco - 1].wait()

  return _sc_scatter_body


def _make_sc(boff):
  return functools.partial(
      pl.kernel,
      out_type=jax.ShapeDtypeStruct((NC * B_H * ACC,), jnp.float32),
      mesh=plsc.VectorSubcoreMesh(
          core_axis_name="c", subcore_axis_name="s", num_cores=NC,
          num_subcores=NS),
      scratch_types=[
          pltpu.VMEM_SHARED((B_H * ACC,), jnp.float32),
          pltpu.VMEM((CH,), jnp.float32),
          pltpu.VMEM((CH,), jnp.int32),
          pltpu.VMEM((CH,), jnp.float32),
          pltpu.VMEM((CH,), jnp.int32),
          pltpu.VMEM((NSUB, 4 * SUB), jnp.int32),
          pltpu.VMEM((NSUB, 4 * SUB), jnp.float32),
          pltpu.VMEM((NSUB, 4 * SUB), jnp.int32),
          pltpu.VMEM((NSUB, 4 * SUB), jnp.float32),
          pltpu.VMEM((ZB,), jnp.float32),
          pltpu.VMEM((ZB,), jnp.float32),
          pltpu.SemaphoreType.DMA,
          pltpu.SemaphoreType.DMA,
          pltpu.SemaphoreType.DMA,
          pltpu.SemaphoreType.DMA,
          pltpu.SemaphoreType.DMA,
          pltpu.SemaphoreType.DMA,
      ],
  )(_make_sc_body(boff))


_sc_scatter_lo = _make_sc(0)
_sc_scatter_hi = _make_sc(B_H)


def _tc_cumsum_body(p0_ref, p1_ref, o_ref):
  d = (p0_ref[...] + p1_ref[...]).reshape(ROWS_P, W).astype(jnp.bfloat16)
  rows_i = lax.broadcasted_iota(jnp.int32, (FEAT, ROWS_P), 0)
  cols_i = lax.broadcasted_iota(jnp.int32, (FEAT, ROWS_P), 1)
  ltri = (rows_i >= cols_i).astype(jnp.bfloat16)
  c1 = jnp.dot(ltri, d, preferred_element_type=jnp.float32)
  xs_i = lax.broadcasted_iota(jnp.int32, (W, FEAT), 0)
  js_i = lax.broadcasted_iota(jnp.int32, (W, FEAT), 1)
  utri = (xs_i <= js_i).astype(jnp.bfloat16)
  c2 = jnp.dot(c1.astype(jnp.bfloat16), utri,
               preferred_element_type=jnp.float32)
  o_ref[0] = 1.0 / (1.0 + jnp.exp(-c2))


def _tc_cumsum(parts):
  return pl.pallas_call(
      _tc_cumsum_body,
      grid=(B_H,),
      in_specs=[
          pl.BlockSpec((ACC,), lambda b: (b,)),
          pl.BlockSpec((ACC,), lambda b: (B_H + b,)),
      ],
      out_specs=pl.BlockSpec((1, FEAT, FEAT), lambda b: (b, 0, 0)),
      out_shape=jax.ShapeDtypeStruct((B_H, FEAT, FEAT), jnp.float32),
  )(parts, parts)


def kernel(preds):
  bb = jnp.clip((preds[:, :, 3:7] * FEAT).astype(jnp.int32), 0, FEAT)
  x1i, y1i = bb[:, :, 0], bb[:, :, 1]
  x2i = jnp.maximum(bb[:, :, 2], x1i)
  y2i = jnp.maximum(bb[:, :, 3], y1i)
  codes = x1i + (y1i << 8) + (x2i << 16) + (y2i << 24)
  conf = preds[:, :, 0]
  conf = jnp.pad(conf, ((0, 0), (0, NPAD - N))).reshape(-1)
  codes = jnp.pad(codes, ((0, 0), (0, NPAD - N))).reshape(-1)
  parts_lo = _sc_scatter_lo(conf, codes)
  parts_hi = _sc_scatter_hi(conf, codes)
  return jnp.concatenate([_tc_cumsum(parts_lo), _tc_cumsum(parts_hi)], axis=0)


# speedup vs baseline: 1.1219x; 1.1219x over previous
import functools

import jax
import jax.numpy as jnp
from jax import lax
from jax.experimental import pallas as pl
from jax.experimental.pallas import tpu as pltpu
from jax.experimental.pallas import tpu_sc as plsc

B = 16
N = 20000
FEAT = 224
W = 256
HROW = 225
ROWS_P = 232
ACC = ROWS_P * W
NC = 2
NS = 16
NPAD = 20480
NWBOX = NPAD // NC
CH = 256
NCHUNK = NWBOX // CH
SUB = 32
NSUB = CH // SUB
ZB = ACC // 8


def _sc_scatter_body(conf, codes, out, acc, cb_a, qb_a, cb_b, qb_b, ib_a,
                     vb_a, ib_b, vb_b, zb_a, zb_b, semi_a, semi_b, sems_a,
                     sems_b, sem_z, sem_o):
  c = lax.axis_index("c")
  s = lax.axis_index("s")
  soff = s * ACC
  base = s * NPAD + c * NWBOX
  m8 = jnp.int32(255)

  def _stage(t, cbr, qbr, sem):
    st = base + t * CH
    pltpu.async_copy(conf.at[pl.ds(st, CH)], cbr, sem)
    pltpu.async_copy(codes.at[pl.ds(st, CH)], qbr, sem)

  def _drain_stage(cbr, qbr, sem):
    pltpu.make_async_copy(conf.at[pl.ds(0, CH)], cbr, sem).wait()
    pltpu.make_async_copy(codes.at[pl.ds(0, CH)], qbr, sem).wait()

  def _fill(cbr, qbr, ibr, vbr):
    for j in range(NSUB):
      for g in range(SUB // 16):
        og = j * SUB + g * 16
        cf = cbr[pl.ds(og, 16)]
        q = qbr[pl.ds(og, 16)]
        xi1 = q & m8
        yi1 = (q >> 8) & m8
        xi2 = (q >> 16) & m8
        yi2 = (q >> 24) & m8
        r1 = soff + yi1 * W
        r2 = soff + yi2 * W
        off = g * 64
        ibr[j, pl.ds(off, 16)] = r1 + xi1
        ibr[j, pl.ds(off + 16, 16)] = r1 + xi2
        ibr[j, pl.ds(off + 32, 16)] = r2 + xi1
        ibr[j, pl.ds(off + 48, 16)] = r2 + xi2
        ncf = -cf
        vbr[j, pl.ds(off, 16)] = cf
        vbr[j, pl.ds(off + 16, 16)] = ncf
        vbr[j, pl.ds(off + 32, 16)] = ncf
        vbr[j, pl.ds(off + 48, 16)] = cf

  def _fire(ibr, vbr, sem):
    for j in range(NSUB):
      pltpu.async_copy(vbr.at[j], acc.at[ibr.at[j]], sem, add=True)

  def _drain_scat(vbr, sem):
    for j in range(NSUB):
      pltpu.make_async_copy(vbr.at[j], acc.at[pl.ds(0, 4 * SUB)], sem).wait()

  _stage(0, cb_a, qb_a, semi_a)

  def _zb(i, carry):
    zb_a[pl.ds(i * 16, 16)] = jnp.zeros((16,), jnp.float32)
    return carry

  lax.fori_loop(0, ZB // 16, _zb, 0)
  zds = [
      pltpu.async_copy(zb_a, acc.at[pl.ds(soff + k * ZB, ZB)], sem_z)
      for k in range(ACC // ZB)
  ]
  for d in zds:
    d.wait()

  def _pair(u, carry):
    _stage(2 * u + 1, cb_b, qb_b, semi_b)
    _drain_stage(cb_a, qb_a, semi_a)

    @pl.when(u > 0)
    def _():
      _drain_scat(vb_a, sems_a)

    _fill(cb_a, qb_a, ib_a, vb_a)
    _fire(ib_a, vb_a, sems_a)
    _drain_stage(cb_b, qb_b, semi_b)

    @pl.when(u > 0)
    def _():
      _drain_scat(vb_b, sems_b)

    _fill(cb_b, qb_b, ib_b, vb_b)
    _fire(ib_b, vb_b, sems_b)

    @pl.when(u + 1 < NCHUNK // 2)
    def _():
      _stage(2 * u + 2, cb_a, qb_a, semi_a)

    return carry

  lax.fori_loop(0, NCHUNK // 2, _pair, 0)
  _drain_scat(vb_a, sems_a)
  _drain_scat(vb_b, sems_b)

  obase = (c * B + s) * ACC
  nco = ACC // ZB
  zb = (zb_a, zb_b)
  dins = [None] * nco
  douts = [None] * nco
  dins[0] = pltpu.async_copy(acc.at[pl.ds(soff, ZB)], zb_a, sem_z)
  for k in range(nco):
    dins[k].wait()
    douts[k] = pltpu.async_copy(
        zb[k % 2], out.at[pl.ds(obase + k * ZB, ZB)], sem_o)
    if k + 1 < nco:
      if k >= 1:
        douts[k - 1].wait()
      dins[k + 1] = pltpu.async_copy(
          acc.at[pl.ds(soff + (k + 1) * ZB, ZB)], zb[(k + 1) % 2], sem_z)
  douts[nco - 2].wait()
  douts[nco - 1].wait()


_sc_scatter = functools.partial(
    pl.kernel,
    out_type=jax.ShapeDtypeStruct((NC * B * ACC,), jnp.float32),
    mesh=plsc.VectorSubcoreMesh(
        core_axis_name="c", subcore_axis_name="s", num_cores=NC,
        num_subcores=NS),
    scratch_types=[
        pltpu.VMEM_SHARED((B * ACC,), jnp.float32),
        pltpu.VMEM((CH,), jnp.float32),
        pltpu.VMEM((CH,), jnp.int32),
        pltpu.VMEM((CH,), jnp.float32),
        pltpu.VMEM((CH,), jnp.int32),
        pltpu.VMEM((NSUB, 4 * SUB), jnp.int32),
        pltpu.VMEM((NSUB, 4 * SUB), jnp.float32),
        pltpu.VMEM((NSUB, 4 * SUB), jnp.int32),
        pltpu.VMEM((NSUB, 4 * SUB), jnp.float32),
        pltpu.VMEM((ZB,), jnp.float32),
        pltpu.VMEM((ZB,), jnp.float32),
        pltpu.SemaphoreType.DMA,
        pltpu.SemaphoreType.DMA,
        pltpu.SemaphoreType.DMA,
        pltpu.SemaphoreType.DMA,
        pltpu.SemaphoreType.DMA,
        pltpu.SemaphoreType.DMA,
    ],
)(_sc_scatter_body)


def _tc_cumsum_body(p0_ref, p1_ref, o_ref):
  d = (p0_ref[...] + p1_ref[...]).reshape(ROWS_P, W)
  rows_i = lax.broadcasted_iota(jnp.int32, (FEAT, ROWS_P), 0)
  cols_i = lax.broadcasted_iota(jnp.int32, (FEAT, ROWS_P), 1)
  ltri = (rows_i >= cols_i).astype(jnp.float32)
  c1 = jnp.dot(ltri, d, preferred_element_type=jnp.float32)
  xs_i = lax.broadcasted_iota(jnp.int32, (W, FEAT), 0)
  js_i = lax.broadcasted_iota(jnp.int32, (W, FEAT), 1)
  utri = (xs_i <= js_i).astype(jnp.bfloat16)
  c2 = jnp.dot(c1.astype(jnp.bfloat16), utri,
               preferred_element_type=jnp.float32)
  o_ref[0] = 1.0 / (1.0 + jnp.exp(-c2))


def kernel(preds):
  bb = jnp.clip((preds[:, :, 3:7] * FEAT).astype(jnp.int32), 0, FEAT)
  x1i, y1i = bb[:, :, 0], bb[:, :, 1]
  x2i = jnp.maximum(bb[:, :, 2], x1i)
  y2i = jnp.maximum(bb[:, :, 3], y1i)
  codes = x1i + (y1i << 8) + (x2i << 16) + (y2i << 24)
  conf = preds[:, :, 0]
  conf = jnp.pad(conf, ((0, 0), (0, NPAD - N))).reshape(-1)
  codes = jnp.pad(codes, ((0, 0), (0, NPAD - N))).reshape(-1)
  parts = _sc_scatter(conf, codes)
  return pl.pallas_call(
      _tc_cumsum_body,
      grid=(B,),
      in_specs=[
          pl.BlockSpec((ACC,), lambda b: (b,)),
          pl.BlockSpec((ACC,), lambda b: (B + b,)),
      ],
      out_specs=pl.BlockSpec((1, FEAT, FEAT), lambda b: (b, 0, 0)),
      out_shape=jax.ShapeDtypeStruct((B, FEAT, FEAT), jnp.float32),
  )(parts, parts)
